# baseline (speedup 1.0000x reference)
import functools

import jax
import jax.numpy as jnp
from jax import lax
from jax.experimental import pallas as pl
from jax.experimental.pallas import tpu as pltpu
from jax.experimental.pallas import tpu_sc as plsc

F32 = jnp.float32


def _make_seg_sum(N, E, D_HALF, with_deg, K=128, NS=16, GRP=8):
    NCH = E // K
    assert NCH * K == E
    NCH_PAD = ((NCH + 7) // 8) * 8
    NK_HI, NK_LO = 80, 72
    N_HI = 12
    TAIL = NCH - N_HI * NK_HI - (NS - N_HI) * NK_LO
    assert 0 <= TAIL <= 2 and NK_HI % GRP == 0 and NK_LO % GRP == 0
    TAIL_BASE = NCH - TAIL
    assert TAIL_BASE % 8 == 0

    ROWS_PT = ((N // NS) // 8 + 1) * 8
    ROWS_LAST = N - ROWS_PT * (NS - 1)
    assert ROWS_LAST > 0 and ROWS_LAST % 8 == 0
    NZ_FULL, ZR_REM = divmod(ROWS_PT, K)
    NZ_LAST, ZR_REM_L = divmod(ROWS_LAST, K)
    assert ZR_REM % 8 == 0 and ZR_REM_L % 8 == 0

    mesh = plsc.VectorSubcoreMesh(core_axis_name="c", subcore_axis_name="s")

    out_type = [jax.ShapeDtypeStruct((2 * N, D_HALF), F32)]
    if with_deg:
        out_type.append(jax.ShapeDtypeStruct((N,), F32))

    @functools.partial(
        pl.kernel,
        out_type=out_type,
        mesh=mesh,
        scratch_types=[
            pltpu.VMEM_SHARED((N, D_HALF), F32),
            pltpu.VMEM_SHARED((N,), F32),
            pltpu.VMEM((2, GRP, K), jnp.int32),
            pltpu.VMEM((2, GRP, K), jnp.int32),
            pltpu.VMEM((2, K), jnp.int32),
            pltpu.VMEM((2, K), jnp.int32),
            pltpu.VMEM((2, K, D_HALF), F32),
            pltpu.VMEM((K,), F32),
            pltpu.VMEM((640,), F32),
            pltpu.SemaphoreType.DMA,
            pltpu.SemaphoreType.DMA,
            pltpu.SemaphoreType.DMA,
            pltpu.SemaphoreType.DMA,
        ],
    )
    def seg_sum(zt, srct, dst2d, *refs):
        if with_deg:
            (outc, deg_out, acc_sh, deg_sh, sidx_v, didx_v, tsidx_v,
             tdidx_v, rows_v, ones_v, zflat_v, gsem0, gsem1, rsem,
             dsem) = refs
        else:
            (outc, acc_sh, deg_sh, sidx_v, didx_v, tsidx_v,
             tdidx_v, rows_v, ones_v, zflat_v, gsem0, gsem1, rsem,
             dsem) = refs
        gsems = (gsem0, gsem1)
        cid = lax.axis_index("c")
        sid = lax.axis_index("s")
        sbase = cid * NCH_PAD

        zeros16 = jnp.zeros((16,), F32)
        ones16 = jnp.ones((16,), F32)
        lanes = D_HALF // 16

        def _zrows(i, carry):
            rows_v[0, i // lanes, pl.ds((i % lanes) * 16, 16)] = zeros16
            return carry
        lax.fori_loop(0, K * lanes, _zrows, 0)

        if with_deg:
            def _zflat(i, carry):
                zflat_v[pl.ds(i * 16, 16)] = zeros16
                return carry
            lax.fori_loop(0, 640 // 16, _zflat, 0)

            def _ones(i, carry):
                ones_v[pl.ds(i * 16, 16)] = ones16
                return carry
            lax.fori_loop(0, K // 16, _ones, 0)

        nk = jnp.where(sid < N_HI, NK_HI, NK_LO)
        cbase = jnp.where(sid < N_HI, sid * NK_HI,
                          N_HI * NK_HI + (sid - N_HI) * NK_LO)

        @pl.when(sid < TAIL)
        def _():
            pltpu.sync_copy(srct.at[pl.ds(sbase + TAIL_BASE, TAIL)], tsidx_v)
            pltpu.sync_copy(dst2d.at[pl.ds(TAIL_BASE, TAIL)], tdidx_v)

        rbase = sid * ROWS_PT

        def _zacc(j, carry):
            pltpu.sync_copy(rows_v.at[0], acc_sh.at[pl.ds(rbase + j * K, K)])
            return carry

        @pl.when(sid < NS - 1)
        def _():
            lax.fori_loop(0, NZ_FULL, _zacc, 0)
            pltpu.sync_copy(rows_v.at[0].at[pl.ds(0, ZR_REM)],
                            acc_sh.at[pl.ds(rbase + NZ_FULL * K, ZR_REM)])

        @pl.when(sid == NS - 1)
        def _():
            lax.fori_loop(0, NZ_LAST, _zacc, 0)
            pltpu.sync_copy(rows_v.at[0].at[pl.ds(0, ZR_REM_L)],
                            acc_sh.at[pl.ds(rbase + NZ_LAST * K, ZR_REM_L)])

        if with_deg:
            @pl.when(cid == 0)
            def _():
                @pl.when(sid < NS - 1)
                def _():
                    pltpu.sync_copy(zflat_v.at[pl.ds(0, ROWS_PT)],
                                    deg_sh.at[pl.ds(rbase, ROWS_PT)])

                @pl.when(sid == NS - 1)
                def _():
                    pltpu.sync_copy(zflat_v.at[pl.ds(0, ROWS_LAST)],
                                    deg_sh.at[pl.ds(rbase, ROWS_LAST)])

        plsc.subcore_barrier()

        def _refill(g):
            p = lax.rem(g, 2)
            row0 = cbase + g * GRP
            pltpu.sync_copy(srct.at[pl.ds(sbase + row0, GRP)], sidx_v.at[p])
            pltpu.sync_copy(dst2d.at[pl.ds(row0, GRP)], didx_v.at[p])

        def _src_row(g, j):
            return sidx_v.at[lax.rem(g, 2), j]

        def _dst_row(g, j):
            return didx_v.at[lax.rem(g, 2), j]

        ng = nk // GRP
        _refill(jnp.int32(0))

        def _group(g, carry):
            gds = [None] * GRP
            dds = [None] * GRP
            gds[0] = pltpu.async_copy(zt.at[_src_row(g, 0)], rows_v.at[0],
                                      gsems[0])
            @pl.when(g + 1 < ng)
            def _():
                _refill(g + 1)
            for j in range(GRP):
                if j < GRP - 1:
                    gds[j + 1] = pltpu.async_copy(
                        zt.at[_src_row(g, j + 1)], rows_v.at[(j + 1) % 2],
                        gsems[(j + 1) % 2])
                gds[j].wait()
                pltpu.sync_copy(rows_v.at[j % 2],
                                acc_sh.at[_dst_row(g, j)], add=True)
                if with_deg:
                    dref = deg_sh.at[_dst_row(g, j)]
                    box = {}

                    @pl.when(cid == 0)
                    def _(j=j, dref=dref, box=box):
                        if j > 0:
                            dds[j - 1]["d"].wait()
                        box["d"] = pltpu.async_copy(ones_v, dref, dsem,
                                                    add=True)
                    dds[j] = box
            if with_deg:
                @pl.when(cid == 0)
                def _():
                    dds[GRP - 1]["d"].wait()
            return carry
        lax.fori_loop(0, ng, _group, 0, unroll=False)

        @pl.when(sid < TAIL)
        def _():
            pltpu.async_copy(zt.at[tsidx_v.at[sid]], rows_v.at[0],
                             gsems[0]).wait()
            pltpu.sync_copy(rows_v.at[0], acc_sh.at[tdidx_v.at[sid]],
                            add=True)
            if with_deg:
                @pl.when(cid == 0)
                def _():
                    pltpu.sync_copy(ones_v, deg_sh.at[tdidx_v.at[sid]],
                                    add=True)

        plsc.subcore_barrier()

        obase = cid * N + rbase

        @pl.when(sid < NS - 1)
        def _():
            pltpu.sync_copy(acc_sh.at[pl.ds(rbase, ROWS_PT)],
                            outc.at[pl.ds(obase, ROWS_PT)])

        @pl.when(sid == NS - 1)
        def _():
            pltpu.sync_copy(acc_sh.at[pl.ds(rbase, ROWS_LAST)],
                            outc.at[pl.ds(obase, ROWS_LAST)])

        if with_deg:
            @pl.when((cid == 0) & (sid < NS - 1))
            def _():
                pltpu.sync_copy(deg_sh.at[pl.ds(rbase, ROWS_PT)],
                                zflat_v.at[pl.ds(0, ROWS_PT)])
                pltpu.sync_copy(zflat_v.at[pl.ds(0, ROWS_PT)],
                                deg_out.at[pl.ds(rbase, ROWS_PT)])

            @pl.when((cid == 0) & (sid == NS - 1))
            def _():
                pltpu.sync_copy(deg_sh.at[pl.ds(rbase, ROWS_LAST)],
                                zflat_v.at[pl.ds(0, ROWS_LAST)])
                pltpu.sync_copy(zflat_v.at[pl.ds(0, ROWS_LAST)],
                                deg_out.at[pl.ds(rbase, ROWS_LAST)])

    return seg_sum


def _mm2_body(h_ref, wl_ref, wr_ref, b_ref, z_ref, r_ref):
    h = h_ref[...]
    z_ref[...] = jnp.dot(h, wl_ref[...], preferred_element_type=F32)
    r_ref[...] = jnp.dot(h, wr_ref[...], preferred_element_type=F32) + b_ref[...]


def _comb_mm_body(s0_ref, s1_ref, deg_ref, r_ref, wl_ref, wr_ref, b_ref,
                  z_ref, rout_ref, *, relu):
    invd = 1.0 / jnp.maximum(deg_ref[...], 1.0)
    h = jnp.concatenate([s0_ref[...], s1_ref[...]], axis=1) * invd + r_ref[...]
    if relu:
        h = jnp.maximum(h, 0.0)
    z_ref[...] = jnp.dot(h, wl_ref[...], preferred_element_type=F32)
    rout_ref[...] = jnp.dot(h, wr_ref[...], preferred_element_type=F32) + b_ref[...]


def _final_body(s0_ref, s1_ref, deg_ref, r_ref, wfc_ref, bfc_ref, out_ref):
    invd = 1.0 / jnp.maximum(deg_ref[...], 1.0)
    h = jnp.concatenate([s0_ref[...], s1_ref[...]], axis=1) * invd + r_ref[...]
    logits = jnp.dot(h, wfc_ref[...], preferred_element_type=F32) + bfc_ref[...]
    m = jnp.max(logits, axis=-1, keepdims=True)
    lse = jnp.log(jnp.sum(jnp.exp(logits - m), axis=-1, keepdims=True))
    out_ref[...] = logits - m - lse


def kernel(x, edge_index, W_l0, W_r0, b0, W_l1, W_r1, b1, W_fc, b_fc):
    N, D = x.shape
    E = edge_index.shape[1]
    H = W_l0.shape[1]
    C = W_fc.shape[1]
    HALF = H // 2
    BN = 1000
    nblk = N // BN
    NCH = E // 128
    NCH_PAD = ((NCH + 7) // 8) * 8
    assert nblk * BN == N and NCH * 128 == E

    src2d = edge_index[0].reshape(NCH, 128)
    dst2d = edge_index[1].reshape(NCH, 128)
    pad = ((0, NCH_PAD - NCH), (0, 0))
    srct = jnp.concatenate(
        [jnp.pad(src2d, pad), jnp.pad(src2d + N, pad)], axis=0)

    seg_sum_deg = _make_seg_sum(N, E, HALF, with_deg=True)
    seg_sum = _make_seg_sum(N, E, HALF, with_deg=False)

    def dense_spec(two_d=False):
        return pl.BlockSpec((BN, D), (lambda i, c: (i, 0)) if two_d else
                            (lambda i: (i, 0)))

    h_spec = pl.BlockSpec((BN, D), lambda i, c: (i, 0))
    wl_spec = pl.BlockSpec((D, HALF), lambda i, c: (0, c))
    b_spec = pl.BlockSpec((1, HALF), lambda i, c: (0, c))
    z_spec = pl.BlockSpec((BN, HALF), lambda i, c: (c * nblk + i, 0))
    r_spec = pl.BlockSpec((BN, HALF), lambda i, c: (i, c))
    s0_spec = pl.BlockSpec((BN, HALF), lambda i, c: (i, 0))
    s1_spec = pl.BlockSpec((BN, HALF), lambda i, c: (nblk + i, 0))
    deg_spec = pl.BlockSpec((BN, 1), lambda i, c: (i, 0))

    z0, r0 = pl.pallas_call(
        _mm2_body,
        grid=(nblk, 2),
        in_specs=[h_spec, wl_spec, wl_spec, b_spec],
        out_specs=[z_spec, r_spec],
        out_shape=[
            jax.ShapeDtypeStruct((2 * N, HALF), F32),
            jax.ShapeDtypeStruct((N, H), F32),
        ],
    )(x, W_l0, W_r0, b0.reshape(1, H))

    s0, deg = seg_sum_deg(z0, srct, dst2d)
    deg2d = deg.reshape(N, 1)

    z1, r1 = pl.pallas_call(
        functools.partial(_comb_mm_body, relu=True),
        grid=(nblk, 2),
        in_specs=[s0_spec, s1_spec, deg_spec, h_spec, wl_spec, wl_spec,
                  b_spec],
        out_specs=[z_spec, r_spec],
        out_shape=[
            jax.ShapeDtypeStruct((2 * N, HALF), F32),
            jax.ShapeDtypeStruct((N, H), F32),
        ],
    )(s0, s0, deg2d, r0, W_l1, W_r1, b1.reshape(1, H))

    s1 = seg_sum(z1, srct, dst2d)[0]

    out = pl.pallas_call(
        _final_body,
        grid=(nblk,),
        in_specs=[pl.BlockSpec((BN, HALF), lambda i: (i, 0)),
                  pl.BlockSpec((BN, HALF), lambda i: (nblk + i, 0)),
                  pl.BlockSpec((BN, 1), lambda i: (i, 0)),
                  dense_spec(),
                  pl.BlockSpec((H, C), lambda i: (0, 0)),
                  pl.BlockSpec((1, C), lambda i: (0, 0))],
        out_specs=pl.BlockSpec((BN, C), lambda i: (i, 0)),
        out_shape=jax.ShapeDtypeStruct((N, C), F32),
    )(s1, s1, deg2d, r1, W_fc, b_fc.reshape(1, C))

    return out


# speedup vs baseline: 1.0533x; 1.0533x over previous
import functools

import jax
import jax.numpy as jnp
from jax import lax
from jax.experimental import pallas as pl
from jax.experimental.pallas import tpu as pltpu
from jax.experimental.pallas import tpu_sc as plsc

F32 = jnp.float32


def _make_seg_sum(N, E, D_HALF, with_deg, K=128, NS=16, GRP=8):
    NCH = E // K
    assert NCH * K == E
    NCH_PAD = ((NCH + 7) // 8) * 8
    NK_HI, NK_LO = 80, 72
    N_HI = 12
    TAIL = NCH - N_HI * NK_HI - (NS - N_HI) * NK_LO
    assert 0 <= TAIL <= 2 and NK_HI % GRP == 0 and NK_LO % GRP == 0
    TAIL_BASE = NCH - TAIL
    assert TAIL_BASE % 8 == 0

    ROWS_PT = ((N // NS) // 8 + 1) * 8
    ROWS_LAST = N - ROWS_PT * (NS - 1)
    assert ROWS_LAST > 0 and ROWS_LAST % 8 == 0
    NZ_FULL, ZR_REM = divmod(ROWS_PT, K)
    NZ_LAST, ZR_REM_L = divmod(ROWS_LAST, K)
    assert ZR_REM % 8 == 0 and ZR_REM_L % 8 == 0

    mesh = plsc.VectorSubcoreMesh(core_axis_name="c", subcore_axis_name="s")

    out_type = [jax.ShapeDtypeStruct((2 * N, D_HALF), F32)]
    if with_deg:
        out_type.append(jax.ShapeDtypeStruct((N,), F32))

    @functools.partial(
        pl.kernel,
        out_type=out_type,
        mesh=mesh,
        scratch_types=[
            pltpu.VMEM_SHARED((N, D_HALF), F32),
            pltpu.VMEM_SHARED((N,), F32),
            pltpu.VMEM((2, GRP, K), jnp.int32),
            pltpu.VMEM((2, GRP, K), jnp.int32),
            pltpu.VMEM((2, K), jnp.int32),
            pltpu.VMEM((2, K), jnp.int32),
            pltpu.VMEM((2, K, D_HALF), F32),
            pltpu.VMEM((K,), F32),
            pltpu.VMEM((640,), F32),
            pltpu.SemaphoreType.DMA,
            pltpu.SemaphoreType.DMA,
            pltpu.SemaphoreType.DMA,
            pltpu.SemaphoreType.DMA,
        ],
    )
    def seg_sum(zt, srct, dst2d, *refs):
        if with_deg:
            (outc, deg_out, acc_sh, deg_sh, sidx_v, didx_v, tsidx_v,
             tdidx_v, rows_v, ones_v, zflat_v, gsem0, gsem1, rsem,
             dsem) = refs
        else:
            (outc, acc_sh, deg_sh, sidx_v, didx_v, tsidx_v,
             tdidx_v, rows_v, ones_v, zflat_v, gsem0, gsem1, rsem,
             dsem) = refs
        gsems = (gsem0, gsem1)
        cid = lax.axis_index("c")
        sid = lax.axis_index("s")
        sbase = cid * NCH_PAD

        zeros16 = jnp.zeros((16,), F32)
        ones16 = jnp.ones((16,), F32)
        lanes = D_HALF // 16

        def _zrows(i, carry):
            rows_v[0, i // lanes, pl.ds((i % lanes) * 16, 16)] = zeros16
            return carry
        lax.fori_loop(0, K * lanes, _zrows, 0)

        if with_deg:
            def _zflat(i, carry):
                zflat_v[pl.ds(i * 16, 16)] = zeros16
                return carry
            lax.fori_loop(0, 640 // 16, _zflat, 0)

            def _ones(i, carry):
                ones_v[pl.ds(i * 16, 16)] = ones16
                return carry
            lax.fori_loop(0, K // 16, _ones, 0)

        nk = jnp.where(sid < N_HI, NK_HI, NK_LO)
        cbase = jnp.where(sid < N_HI, sid * NK_HI,
                          N_HI * NK_HI + (sid - N_HI) * NK_LO)

        @pl.when(sid < TAIL)
        def _():
            pltpu.sync_copy(srct.at[pl.ds(sbase + TAIL_BASE, TAIL)], tsidx_v)
            pltpu.sync_copy(dst2d.at[pl.ds(TAIL_BASE, TAIL)], tdidx_v)

        rbase = sid * ROWS_PT

        def _zacc(j, carry):
            pltpu.sync_copy(rows_v.at[0], acc_sh.at[pl.ds(rbase + j * K, K)])
            return carry

        @pl.when(sid < NS - 1)
        def _():
            lax.fori_loop(0, NZ_FULL, _zacc, 0)
            pltpu.sync_copy(rows_v.at[0].at[pl.ds(0, ZR_REM)],
                            acc_sh.at[pl.ds(rbase + NZ_FULL * K, ZR_REM)])

        @pl.when(sid == NS - 1)
        def _():
            lax.fori_loop(0, NZ_LAST, _zacc, 0)
            pltpu.sync_copy(rows_v.at[0].at[pl.ds(0, ZR_REM_L)],
                            acc_sh.at[pl.ds(rbase + NZ_LAST * K, ZR_REM_L)])

        if with_deg:
            @pl.when(cid == 0)
            def _():
                @pl.when(sid < NS - 1)
                def _():
                    pltpu.sync_copy(zflat_v.at[pl.ds(0, ROWS_PT)],
                                    deg_sh.at[pl.ds(rbase, ROWS_PT)])

                @pl.when(sid == NS - 1)
                def _():
                    pltpu.sync_copy(zflat_v.at[pl.ds(0, ROWS_LAST)],
                                    deg_sh.at[pl.ds(rbase, ROWS_LAST)])

        plsc.subcore_barrier()

        def _refill(g):
            p = lax.rem(g, 2)
            row0 = cbase + g * GRP
            pltpu.sync_copy(srct.at[pl.ds(sbase + row0, GRP)], sidx_v.at[p])
            pltpu.sync_copy(dst2d.at[pl.ds(row0, GRP)], didx_v.at[p])

        def _src_row(g, j):
            return sidx_v.at[lax.rem(g, 2), j]

        def _dst_row(g, j):
            return didx_v.at[lax.rem(g, 2), j]

        ng = nk // GRP
        _refill(jnp.int32(0))

        def _group(g, carry):
            gds = [None] * GRP
            dds = [None] * GRP
            gds[0] = pltpu.async_copy(zt.at[_src_row(g, 0)], rows_v.at[0],
                                      gsems[0])
            @pl.when(g + 1 < ng)
            def _():
                _refill(g + 1)
            for j in range(GRP):
                if j < GRP - 1:
                    gds[j + 1] = pltpu.async_copy(
                        zt.at[_src_row(g, j + 1)], rows_v.at[(j + 1) % 2],
                        gsems[(j + 1) % 2])
                gds[j].wait()
                pltpu.sync_copy(rows_v.at[j % 2],
                                acc_sh.at[_dst_row(g, j)], add=True)
                if with_deg:
                    dref = deg_sh.at[_dst_row(g, j)]
                    box = {}

                    @pl.when(cid == 0)
                    def _(j=j, dref=dref, box=box):
                        if j > 0:
                            dds[j - 1]["d"].wait()
                        box["d"] = pltpu.async_copy(ones_v, dref, dsem,
                                                    add=True)
                    dds[j] = box
            if with_deg:
                @pl.when(cid == 0)
                def _():
                    dds[GRP - 1]["d"].wait()
            return carry
        lax.fori_loop(0, ng, _group, 0, unroll=False)

        @pl.when(sid < TAIL)
        def _():
            pltpu.async_copy(zt.at[tsidx_v.at[sid]], rows_v.at[0],
                             gsems[0]).wait()
            pltpu.sync_copy(rows_v.at[0], acc_sh.at[tdidx_v.at[sid]],
                            add=True)
            if with_deg:
                @pl.when(cid == 0)
                def _():
                    pltpu.sync_copy(ones_v, deg_sh.at[tdidx_v.at[sid]],
                                    add=True)

        plsc.subcore_barrier()

        obase = cid * N + rbase

        @pl.when(sid < NS - 1)
        def _():
            pltpu.sync_copy(acc_sh.at[pl.ds(rbase, ROWS_PT)],
                            outc.at[pl.ds(obase, ROWS_PT)])

        @pl.when(sid == NS - 1)
        def _():
            pltpu.sync_copy(acc_sh.at[pl.ds(rbase, ROWS_LAST)],
                            outc.at[pl.ds(obase, ROWS_LAST)])

        if with_deg:
            @pl.when((cid == 0) & (sid < NS - 1))
            def _():
                pltpu.sync_copy(deg_sh.at[pl.ds(rbase, ROWS_PT)],
                                zflat_v.at[pl.ds(0, ROWS_PT)])
                pltpu.sync_copy(zflat_v.at[pl.ds(0, ROWS_PT)],
                                deg_out.at[pl.ds(rbase, ROWS_PT)])

            @pl.when((cid == 0) & (sid == NS - 1))
            def _():
                pltpu.sync_copy(deg_sh.at[pl.ds(rbase, ROWS_LAST)],
                                zflat_v.at[pl.ds(0, ROWS_LAST)])
                pltpu.sync_copy(zflat_v.at[pl.ds(0, ROWS_LAST)],
                                deg_out.at[pl.ds(rbase, ROWS_LAST)])

    return seg_sum


def _mm_body(h_ref, w_ref, z_ref):
    z_ref[...] = jnp.dot(h_ref[...], w_ref[...], preferred_element_type=F32)


def _mmb_body(h_ref, w_ref, b_ref, r_ref):
    r_ref[...] = jnp.dot(h_ref[...], w_ref[...],
                         preferred_element_type=F32) + b_ref[...]


def _comb_mm_body(s0_ref, s1_ref, deg_ref, r_ref, wl_ref, z_ref, h_ref):
    invd = 1.0 / jnp.maximum(deg_ref[...], 1.0)
    h = jnp.concatenate([s0_ref[...], s1_ref[...]], axis=1) * invd + r_ref[...]
    h = jnp.maximum(h, 0.0)
    z_ref[...] = jnp.dot(h, wl_ref[...], preferred_element_type=F32)
    half = z_ref.shape[1]
    c = pl.program_id(1)
    h_ref[...] = jnp.where(c == 0, h[:, :half], h[:, half:])


def _final_body(s0_ref, s1_ref, deg_ref, r_ref, wfc_ref, bfc_ref, out_ref):
    invd = 1.0 / jnp.maximum(deg_ref[...], 1.0)
    h = jnp.concatenate([s0_ref[...], s1_ref[...]], axis=1) * invd + r_ref[...]
    logits = jnp.dot(h, wfc_ref[...], preferred_element_type=F32) + bfc_ref[...]
    m = jnp.max(logits, axis=-1, keepdims=True)
    lse = jnp.log(jnp.sum(jnp.exp(logits - m), axis=-1, keepdims=True))
    out_ref[...] = logits - m - lse


def kernel(x, edge_index, W_l0, W_r0, b0, W_l1, W_r1, b1, W_fc, b_fc):
    N, D = x.shape
    E = edge_index.shape[1]
    H = W_l0.shape[1]
    C = W_fc.shape[1]
    HALF = H // 2
    BN = 2000
    nblk = N // BN
    NCH = E // 128
    NCH_PAD = ((NCH + 7) // 8) * 8
    assert nblk * BN == N and NCH * 128 == E

    src2d = edge_index[0].reshape(NCH, 128)
    dst2d = edge_index[1].reshape(NCH, 128)
    pad = ((0, NCH_PAD - NCH), (0, 0))
    srct = jnp.concatenate(
        [jnp.pad(src2d, pad), jnp.pad(src2d + N, pad)], axis=0)

    seg_sum_deg = _make_seg_sum(N, E, HALF, with_deg=True)
    seg_sum = _make_seg_sum(N, E, HALF, with_deg=False)

    def dense_spec(two_d=False):
        return pl.BlockSpec((BN, D), (lambda i, c: (i, 0)) if two_d else
                            (lambda i: (i, 0)))

    h_spec = pl.BlockSpec((BN, D), lambda i, c: (i, 0))
    wl_spec = pl.BlockSpec((D, HALF), lambda i, c: (0, c))
    b_spec = pl.BlockSpec((1, HALF), lambda i, c: (0, c))
    z_spec = pl.BlockSpec((BN, HALF), lambda i, c: (c * nblk + i, 0))
    r_spec = pl.BlockSpec((BN, HALF), lambda i, c: (i, c))
    s0_spec = pl.BlockSpec((BN, HALF), lambda i, c: (i, 0))
    s1_spec = pl.BlockSpec((BN, HALF), lambda i, c: (nblk + i, 0))
    deg_spec = pl.BlockSpec((BN, 1), lambda i, c: (i, 0))

    z0 = pl.pallas_call(
        _mm_body,
        grid=(nblk, 2),
        in_specs=[h_spec, wl_spec],
        out_specs=z_spec,
        out_shape=jax.ShapeDtypeStruct((2 * N, HALF), F32),
    )(x, W_l0)

    r0 = pl.pallas_call(
        _mmb_body,
        grid=(nblk, 2),
        in_specs=[h_spec, wl_spec, b_spec],
        out_specs=r_spec,
        out_shape=jax.ShapeDtypeStruct((N, H), F32),
    )(x, W_r0, b0.reshape(1, H))

    s0, deg = seg_sum_deg(z0, srct, dst2d)
    deg2d = deg.reshape(N, 1)

    z1, h1 = pl.pallas_call(
        _comb_mm_body,
        grid=(nblk, 2),
        in_specs=[s0_spec, s1_spec, deg_spec, h_spec, wl_spec],
        out_specs=[z_spec, r_spec],
        out_shape=[
            jax.ShapeDtypeStruct((2 * N, HALF), F32),
            jax.ShapeDtypeStruct((N, H), F32),
        ],
    )(s0, s0, deg2d, r0, W_l1)

    r1 = pl.pallas_call(
        _mmb_body,
        grid=(nblk, 2),
        in_specs=[h_spec, wl_spec, b_spec],
        out_specs=r_spec,
        out_shape=jax.ShapeDtypeStruct((N, H), F32),
    )(h1, W_r1, b1.reshape(1, H))

    s1 = seg_sum(z1, srct, dst2d)[0]

    out = pl.pallas_call(
        _final_body,
        grid=(nblk,),
        in_specs=[pl.BlockSpec((BN, HALF), lambda i: (i, 0)),
                  pl.BlockSpec((BN, HALF), lambda i: (nblk + i, 0)),
                  pl.BlockSpec((BN, 1), lambda i: (i, 0)),
                  dense_spec(),
                  pl.BlockSpec((H, C), lambda i: (0, 0)),
                  pl.BlockSpec((1, C), lambda i: (0, 0))],
        out_specs=pl.BlockSpec((BN, C), lambda i: (i, 0)),
        out_shape=jax.ShapeDtypeStruct((N, C), F32),
    )(s1, s1, deg2d, r1, W_fc, b_fc.reshape(1, C))

    return out


# speedup vs baseline: 1.0552x; 1.0018x over previous
import functools

import jax
import jax.numpy as jnp
from jax import lax
from jax.experimental import pallas as pl
from jax.experimental.pallas import tpu as pltpu
from jax.experimental.pallas import tpu_sc as plsc

F32 = jnp.float32


def _make_seg_sum(N, E, D_HALF, with_deg, K=128, NS=16, GRP=8):
    NCH = E // K
    assert NCH * K == E
    NCH_PAD = ((NCH + 7) // 8) * 8
    NK_HI, NK_LO = 80, 72
    N_HI = 12
    TAIL = NCH - N_HI * NK_HI - (NS - N_HI) * NK_LO
    assert 0 <= TAIL <= 2 and NK_HI % GRP == 0 and NK_LO % GRP == 0
    TAIL_BASE = NCH - TAIL
    assert TAIL_BASE % 8 == 0

    ROWS_PT = ((N // NS) // 8 + 1) * 8
    ROWS_LAST = N - ROWS_PT * (NS - 1)
    assert ROWS_LAST > 0 and ROWS_LAST % 8 == 0
    NZ_FULL, ZR_REM = divmod(ROWS_PT, K)
    NZ_LAST, ZR_REM_L = divmod(ROWS_LAST, K)
    assert ZR_REM % 8 == 0 and ZR_REM_L % 8 == 0

    mesh = plsc.VectorSubcoreMesh(core_axis_name="c", subcore_axis_name="s")

    out_type = [jax.ShapeDtypeStruct((2 * N, D_HALF), F32)]
    if with_deg:
        out_type.append(jax.ShapeDtypeStruct((N,), F32))

    @functools.partial(
        pl.kernel,
        out_type=out_type,
        mesh=mesh,
        scratch_types=[
            pltpu.VMEM_SHARED((N, D_HALF), F32),
            pltpu.VMEM_SHARED((N,), F32),
            pltpu.VMEM((2, GRP, K), jnp.int32),
            pltpu.VMEM((2, GRP, K), jnp.int32),
            pltpu.VMEM((2, K), jnp.int32),
            pltpu.VMEM((2, K), jnp.int32),
            pltpu.VMEM((2, K, D_HALF), F32),
            pltpu.VMEM((K,), F32),
            pltpu.VMEM((640,), F32),
            pltpu.SemaphoreType.DMA,
            pltpu.SemaphoreType.DMA,
            pltpu.SemaphoreType.DMA,
        ],
    )
    def seg_sum(zt, srct, dst2d, *refs):
        if with_deg:
            (outc, deg_out, acc_sh, deg_sh, sidx_v, didx_v, tsidx_v,
             tdidx_v, rows_v, ones_v, zflat_v, gsem0, gsem1, dsem) = refs
        else:
            (outc, acc_sh, deg_sh, sidx_v, didx_v, tsidx_v,
             tdidx_v, rows_v, ones_v, zflat_v, gsem0, gsem1, dsem) = refs
        gsems = (gsem0, gsem1)
        cid = lax.axis_index("c")
        sid = lax.axis_index("s")
        sbase = cid * NCH_PAD

        zeros16 = jnp.zeros((16,), F32)
        ones16 = jnp.ones((16,), F32)
        lanes = D_HALF // 16

        def _zrows(i, carry):
            rows_v[0, i // lanes, pl.ds((i % lanes) * 16, 16)] = zeros16
            return carry
        lax.fori_loop(0, K * lanes, _zrows, 0)

        if with_deg:
            def _zflat(i, carry):
                zflat_v[pl.ds(i * 16, 16)] = zeros16
                return carry
            lax.fori_loop(0, 640 // 16, _zflat, 0)

            def _ones(i, carry):
                ones_v[pl.ds(i * 16, 16)] = ones16
                return carry
            lax.fori_loop(0, K // 16, _ones, 0)

        nk = jnp.where(sid < N_HI, NK_HI, NK_LO)
        cbase = jnp.where(sid < N_HI, sid * NK_HI,
                          N_HI * NK_HI + (sid - N_HI) * NK_LO)

        @pl.when(sid < TAIL)
        def _():
            pltpu.sync_copy(srct.at[pl.ds(sbase + TAIL_BASE, TAIL)], tsidx_v)
            pltpu.sync_copy(dst2d.at[pl.ds(TAIL_BASE, TAIL)], tdidx_v)

        rbase = sid * ROWS_PT

        def _zacc(j, carry):
            pltpu.sync_copy(rows_v.at[0], acc_sh.at[pl.ds(rbase + j * K, K)])
            return carry

        @pl.when(sid < NS - 1)
        def _():
            lax.fori_loop(0, NZ_FULL, _zacc, 0)
            pltpu.sync_copy(rows_v.at[0].at[pl.ds(0, ZR_REM)],
                            acc_sh.at[pl.ds(rbase + NZ_FULL * K, ZR_REM)])

        @pl.when(sid == NS - 1)
        def _():
            lax.fori_loop(0, NZ_LAST, _zacc, 0)
            pltpu.sync_copy(rows_v.at[0].at[pl.ds(0, ZR_REM_L)],
                            acc_sh.at[pl.ds(rbase + NZ_LAST * K, ZR_REM_L)])

        if with_deg:
            @pl.when(cid == 0)
            def _():
                @pl.when(sid < NS - 1)
                def _():
                    pltpu.sync_copy(zflat_v.at[pl.ds(0, ROWS_PT)],
                                    deg_sh.at[pl.ds(rbase, ROWS_PT)])

                @pl.when(sid == NS - 1)
                def _():
                    pltpu.sync_copy(zflat_v.at[pl.ds(0, ROWS_LAST)],
                                    deg_sh.at[pl.ds(rbase, ROWS_LAST)])

        plsc.subcore_barrier()

        def _refill(g):
            p = lax.rem(g, 2)
            row0 = cbase + g * GRP
            pltpu.sync_copy(srct.at[pl.ds(sbase + row0, GRP)], sidx_v.at[p])
            pltpu.sync_copy(dst2d.at[pl.ds(row0, GRP)], didx_v.at[p])

        def _src_row(g, j):
            return sidx_v.at[lax.rem(g, 2), j]

        def _dst_row(g, j):
            return didx_v.at[lax.rem(g, 2), j]

        ng = nk // GRP
        _refill(jnp.int32(0))

        def _group(g, carry):
            gds = [None] * GRP
            dds = [None] * GRP
            gds[0] = pltpu.async_copy(zt.at[_src_row(g, 0)], rows_v.at[0],
                                      gsems[0])
            @pl.when(g + 1 < ng)
            def _():
                _refill(g + 1)
            for j in range(GRP):
                if j < GRP - 1:
                    gds[j + 1] = pltpu.async_copy(
                        zt.at[_src_row(g, j + 1)], rows_v.at[(j + 1) % 2],
                        gsems[(j + 1) % 2])
                gds[j].wait()
                pltpu.sync_copy(rows_v.at[j % 2],
                                acc_sh.at[_dst_row(g, j)], add=True)
                if with_deg:
                    dref = deg_sh.at[_dst_row(g, j)]
                    box = {}

                    @pl.when(cid == 0)
                    def _(j=j, dref=dref, box=box):
                        if j > 0:
                            dds[j - 1]["d"].wait()
                        box["d"] = pltpu.async_copy(ones_v, dref, dsem,
                                                    add=True)
                    dds[j] = box
            if with_deg:
                @pl.when(cid == 0)
                def _():
                    dds[GRP - 1]["d"].wait()
            return carry
        lax.fori_loop(0, ng, _group, 0, unroll=False)

        @pl.when(sid < TAIL)
        def _():
            pltpu.async_copy(zt.at[tsidx_v.at[sid]], rows_v.at[0],
                             gsems[0]).wait()
            pltpu.sync_copy(rows_v.at[0], acc_sh.at[tdidx_v.at[sid]],
                            add=True)
            if with_deg:
                @pl.when(cid == 0)
                def _():
                    pltpu.sync_copy(ones_v, deg_sh.at[tdidx_v.at[sid]],
                                    add=True)

        plsc.subcore_barrier()

        obase = cid * N + rbase

        @pl.when(sid < NS - 1)
        def _():
            pltpu.sync_copy(acc_sh.at[pl.ds(rbase, ROWS_PT)],
                            outc.at[pl.ds(obase, ROWS_PT)])

        @pl.when(sid == NS - 1)
        def _():
            pltpu.sync_copy(acc_sh.at[pl.ds(rbase, ROWS_LAST)],
                            outc.at[pl.ds(obase, ROWS_LAST)])

        if with_deg:
            @pl.when((cid == 0) & (sid < NS - 1))
            def _():
                pltpu.sync_copy(deg_sh.at[pl.ds(rbase, ROWS_PT)],
                                zflat_v.at[pl.ds(0, ROWS_PT)])
                pltpu.sync_copy(zflat_v.at[pl.ds(0, ROWS_PT)],
                                deg_out.at[pl.ds(rbase, ROWS_PT)])

            @pl.when((cid == 0) & (sid == NS - 1))
            def _():
                pltpu.sync_copy(deg_sh.at[pl.ds(rbase, ROWS_LAST)],
                                zflat_v.at[pl.ds(0, ROWS_LAST)])
                pltpu.sync_copy(zflat_v.at[pl.ds(0, ROWS_LAST)],
                                deg_out.at[pl.ds(rbase, ROWS_LAST)])

    return seg_sum


def _mm_body(h_ref, w_ref, z_ref):
    z_ref[...] = jnp.dot(h_ref[...], w_ref[...], preferred_element_type=F32)


def _mmb_body(h_ref, w_ref, b_ref, r_ref):
    r_ref[...] = jnp.dot(h_ref[...], w_ref[...],
                         preferred_element_type=F32) + b_ref[...]


def _comb_mm_body(s0_ref, s1_ref, deg_ref, r_ref, wl_ref, z_ref, h_ref):
    invd = 1.0 / jnp.maximum(deg_ref[...], 1.0)
    h = jnp.concatenate([s0_ref[...], s1_ref[...]], axis=1) * invd + r_ref[...]
    h = jnp.maximum(h, 0.0)
    z_ref[...] = jnp.dot(h, wl_ref[...], preferred_element_type=F32)
    half = z_ref.shape[1]
    c = pl.program_id(1)
    h_ref[...] = jnp.where(c == 0, h[:, :half], h[:, half:])


def _final_body(s0_ref, s1_ref, deg_ref, r_ref, wfc_ref, bfc_ref, out_ref):
    invd = 1.0 / jnp.maximum(deg_ref[...], 1.0)
    h = jnp.concatenate([s0_ref[...], s1_ref[...]], axis=1) * invd + r_ref[...]
    logits = jnp.dot(h, wfc_ref[...], preferred_element_type=F32) + bfc_ref[...]
    m = jnp.max(logits, axis=-1, keepdims=True)
    lse = jnp.log(jnp.sum(jnp.exp(logits - m), axis=-1, keepdims=True))
    out_ref[...] = logits - m - lse


def kernel(x, edge_index, W_l0, W_r0, b0, W_l1, W_r1, b1, W_fc, b_fc):
    N, D = x.shape
    E = edge_index.shape[1]
    H = W_l0.shape[1]
    C = W_fc.shape[1]
    HALF = H // 2
    BN = 2000
    nblk = N // BN
    NCH = E // 128
    NCH_PAD = ((NCH + 7) // 8) * 8
    assert nblk * BN == N and NCH * 128 == E

    src2d = edge_index[0].reshape(NCH, 128)
    dst2d = edge_index[1].reshape(NCH, 128)
    pad = ((0, NCH_PAD - NCH), (0, 0))
    srct = jnp.concatenate(
        [jnp.pad(src2d, pad), jnp.pad(src2d + N, pad)], axis=0)

    seg_sum_deg = _make_seg_sum(N, E, HALF, with_deg=True)
    seg_sum = _make_seg_sum(N, E, HALF, with_deg=False)

    def dense_spec(two_d=False):
        return pl.BlockSpec((BN, D), (lambda i, c: (i, 0)) if two_d else
                            (lambda i: (i, 0)))

    h_spec = pl.BlockSpec((BN, D), lambda i, c: (i, 0))
    wl_spec = pl.BlockSpec((D, HALF), lambda i, c: (0, c))
    b_spec = pl.BlockSpec((1, HALF), lambda i, c: (0, c))
    z_spec = pl.BlockSpec((BN, HALF), lambda i, c: (c * nblk + i, 0))
    r_spec = pl.BlockSpec((BN, HALF), lambda i, c: (i, c))
    s0_spec = pl.BlockSpec((BN, HALF), lambda i, c: (i, 0))
    s1_spec = pl.BlockSpec((BN, HALF), lambda i, c: (nblk + i, 0))
    deg_spec = pl.BlockSpec((BN, 1), lambda i, c: (i, 0))

    z0 = pl.pallas_call(
        _mm_body,
        grid=(nblk, 2),
        in_specs=[h_spec, wl_spec],
        out_specs=z_spec,
        out_shape=jax.ShapeDtypeStruct((2 * N, HALF), F32),
    )(x, W_l0)

    r0 = pl.pallas_call(
        _mmb_body,
        grid=(nblk, 2),
        in_specs=[h_spec, wl_spec, b_spec],
        out_specs=r_spec,
        out_shape=jax.ShapeDtypeStruct((N, H), F32),
    )(x, W_r0, b0.reshape(1, H))

    s0, deg = seg_sum_deg(z0, srct, dst2d)
    deg2d = deg.reshape(N, 1)

    z1, h1 = pl.pallas_call(
        _comb_mm_body,
        grid=(nblk, 2),
        in_specs=[s0_spec, s1_spec, deg_spec, h_spec, wl_spec],
        out_specs=[z_spec, r_spec],
        out_shape=[
            jax.ShapeDtypeStruct((2 * N, HALF), F32),
            jax.ShapeDtypeStruct((N, H), F32),
        ],
    )(s0, s0, deg2d, r0, W_l1)

    r1 = pl.pallas_call(
        _mmb_body,
        grid=(nblk, 2),
        in_specs=[h_spec, wl_spec, b_spec],
        out_specs=r_spec,
        out_shape=jax.ShapeDtypeStruct((N, H), F32),
    )(h1, W_r1, b1.reshape(1, H))

    s1 = seg_sum(z1, srct, dst2d)[0]

    out = pl.pallas_call(
        _final_body,
        grid=(nblk,),
        in_specs=[pl.BlockSpec((BN, HALF), lambda i: (i, 0)),
                  pl.BlockSpec((BN, HALF), lambda i: (nblk + i, 0)),
                  pl.BlockSpec((BN, 1), lambda i: (i, 0)),
                  dense_spec(),
                  pl.BlockSpec((H, C), lambda i: (0, 0)),
                  pl.BlockSpec((1, C), lambda i: (0, 0))],
        out_specs=pl.BlockSpec((BN, C), lambda i: (i, 0)),
        out_shape=jax.ShapeDtypeStruct((N, C), F32),
    )(s1, s1, deg2d, r1, W_fc, b_fc.reshape(1, C))

    return out
